# baseline (speedup 1.0000x reference)
import functools

import jax
import jax.numpy as jnp
from jax import lax
from jax.experimental import pallas as pl
from jax.experimental.pallas import tpu as pltpu
from jax.experimental.pallas import tpu_sc as plsc

NC = 2
NS = 16
NW = NC * NS
C = 512


def _make_gather(nchunk: int, d: int):
    mesh = plsc.VectorSubcoreMesh(
        core_axis_name="c", subcore_axis_name="s", num_cores=NC, num_subcores=NS
    )
    b_per_w = nchunk * C
    total = NW * b_per_w

    @functools.partial(
        pl.kernel,
        out_type=jax.ShapeDtypeStruct((total, d), jnp.float32),
        mesh=mesh,
        compiler_params=pltpu.CompilerParams(use_tc_tiling_on_sc=False),
        scratch_types=[
            pltpu.VMEM((C,), jnp.int32),
            pltpu.VMEM((C,), jnp.int32),
            pltpu.VMEM((C, d), jnp.float32),
            pltpu.VMEM((C, d), jnp.float32),
            pltpu.SemaphoreType.DMA,
            pltpu.SemaphoreType.DMA,
        ],
    )
    def emb(idx_hbm, table_hbm, out_hbm, idx0, idx1, rows0, rows1, g0, g1):
        wid = lax.axis_index("s") * NC + lax.axis_index("c")
        base = wid * b_per_w
        idxb = (idx0, idx1)
        rows = (rows0, rows1)
        gsem = (g0, g1)

        for b in range(2):
            pltpu.sync_copy(idx_hbm.at[wid, b], idxb[b])
            pltpu.async_copy(table_hbm.at[idxb[b]], rows[b], gsem[b])

        def body(i, carry):
            for b in range(2):
                jj = 2 * i + b
                pltpu.make_async_copy(
                    table_hbm.at[idxb[b]], rows[b], gsem[b]
                ).wait()
                pltpu.sync_copy(rows[b], out_hbm.at[pl.ds(base + jj * C, C)])
                pltpu.sync_copy(idx_hbm.at[wid, jj + 2], idxb[b])
                pltpu.async_copy(table_hbm.at[idxb[b]], rows[b], gsem[b])
            return carry

        lax.fori_loop(0, nchunk // 2 - 1, body, 0)

        for b in range(2):
            jj = nchunk - 2 + b
            pltpu.make_async_copy(
                table_hbm.at[idxb[b]], rows[b], gsem[b]
            ).wait()
            pltpu.sync_copy(rows[b], out_hbm.at[pl.ds(base + jj * C, C)])

    return emb


def kernel(obs, table):
    batch, fields = obs.shape
    num_in, d = table.shape
    total = batch * fields
    flat = obs.reshape(total).astype(jnp.int32)

    stride = NW * C
    padded = ((total + stride - 1) // stride) * stride
    if padded != total:
        flat = jnp.concatenate(
            [flat, jnp.zeros((padded - total,), dtype=jnp.int32)]
        )
    nchunk = padded // stride
    idx = flat.reshape(NW, nchunk, C)

    out = _make_gather(nchunk, d)(idx, table)
    if padded != total:
        out = out[:total]
    return out.reshape(batch, fields, d)


# speedup vs baseline: 1.0091x; 1.0091x over previous
import functools

import jax
import jax.numpy as jnp
from jax import lax
from jax.experimental import pallas as pl
from jax.experimental.pallas import tpu as pltpu
from jax.experimental.pallas import tpu_sc as plsc

NC = 2
NS = 16
NW = NC * NS
C = 416
NBUF = 4


def _make_gather(nchunk: int, d: int):
    mesh = plsc.VectorSubcoreMesh(
        core_axis_name="c", subcore_axis_name="s", num_cores=NC, num_subcores=NS
    )
    b_per_w = nchunk * C
    total = NW * b_per_w

    @functools.partial(
        pl.kernel,
        out_type=jax.ShapeDtypeStruct((total, d), jnp.float32),
        mesh=mesh,
        compiler_params=pltpu.CompilerParams(use_tc_tiling_on_sc=False),
        scratch_types=[
            pltpu.VMEM((b_per_w,), jnp.int32),
            [pltpu.VMEM((C, d), jnp.float32) for _ in range(NBUF)],
            [pltpu.SemaphoreType.DMA for _ in range(NBUF)],
        ],
    )
    def emb(idx_hbm, table_hbm, out_hbm, idx_v, rows, gsem):
        wid = lax.axis_index("s") * NC + lax.axis_index("c")
        base = wid * b_per_w
        pltpu.sync_copy(idx_hbm.at[wid], idx_v)

        for b in range(NBUF):
            pltpu.async_copy(
                table_hbm.at[idx_v.at[pl.ds(b * C, C)]], rows[b], gsem[b]
            )

        def body(i, carry):
            for b in range(NBUF):
                jj = NBUF * i + b
                pltpu.make_async_copy(
                    table_hbm.at[idx_v.at[pl.ds(0, C)]], rows[b], gsem[b]
                ).wait()
                pltpu.sync_copy(rows[b], out_hbm.at[pl.ds(base + jj * C, C)])
                pltpu.async_copy(
                    table_hbm.at[idx_v.at[pl.ds((jj + NBUF) * C, C)]],
                    rows[b],
                    gsem[b],
                )
            return carry

        lax.fori_loop(0, nchunk // NBUF - 1, body, 0)

        for b in range(NBUF):
            jj = nchunk - NBUF + b
            pltpu.make_async_copy(
                table_hbm.at[idx_v.at[pl.ds(0, C)]], rows[b], gsem[b]
            ).wait()
            pltpu.sync_copy(rows[b], out_hbm.at[pl.ds(base + jj * C, C)])

    return emb


def kernel(obs, table):
    batch, fields = obs.shape
    num_in, d = table.shape
    total = batch * fields
    flat = obs.reshape(total).astype(jnp.int32)

    stride = NW * C * NBUF
    padded = ((total + stride - 1) // stride) * stride
    if padded != total:
        flat = jnp.concatenate(
            [flat, jnp.zeros((padded - total,), dtype=jnp.int32)]
        )
    nchunk = padded // (NW * C)
    idx = flat.reshape(NW, nchunk * C)

    out = _make_gather(nchunk, d)(idx, table)
    if padded != total:
        out = out[:total]
    return out.reshape(batch, fields, d)
